# Initial kernel scaffold; baseline (speedup 1.0000x reference)
import jax
import jax.numpy as jnp
from jax.experimental import pallas as pl


def kernel(x, W, b, edge_index):
    raise NotImplementedError("write your pallas kernel here")


# speedup vs baseline: 13.9743x; 13.9743x over previous
import functools

import jax
import jax.numpy as jnp
from jax import lax
from jax.experimental import pallas as pl
from jax.experimental.pallas import tpu as pltpu
from jax.experimental.pallas import tpu_sc as plsc

N = 10000
E = 320000
D = 128
DH = 64
NC = 2
NS = 16
EPT = E // NS
C = 80
NPH = 2
NCH = EPT // (C * NPH)
RING = 5
RITER = NCH // RING
NZC = N // C
ZPT = -(-NZC // NS)

ROW_BLK = 1000


def _mlp_body(x_ref, wt_ref, b_ref, haug_ref):
    h = jnp.dot(x_ref[...], wt_ref[...], preferred_element_type=jnp.float32)
    h = jnp.maximum(h + b_ref[...], 0.0)
    n2 = jnp.sum(h * h, axis=1, keepdims=True)
    nrm = jnp.sqrt(n2)
    h = h / jnp.maximum(nrm, 1e-12)
    haug_ref[0] = h[:, :DH]
    haug_ref[1] = h[:, DH:]


_mlp = pl.pallas_call(
    _mlp_body,
    grid=(N // ROW_BLK,),
    in_specs=[
        pl.BlockSpec((ROW_BLK, D), lambda i: (i, 0)),
        pl.BlockSpec((D, D), lambda i: (0, 0)),
        pl.BlockSpec((1, D), lambda i: (0, 0)),
    ],
    out_specs=pl.BlockSpec((NC, ROW_BLK, DH), lambda i: (0, i, 0)),
    out_shape=jax.ShapeDtypeStruct((NC, N, DH), jnp.float32),
)


_sc_mesh = plsc.VectorSubcoreMesh(core_axis_name="c", subcore_axis_name="s")
_sc_params = pltpu.CompilerParams(
    use_tc_tiling_on_sc=False, needs_layout_passes=False
)


@functools.partial(
    pl.kernel,
    out_type=jax.ShapeDtypeStruct((N, D), jnp.float32),
    mesh=_sc_mesh,
    scratch_types=[
        pltpu.VMEM((NCH, C), jnp.int32),
        pltpu.VMEM((NCH, C), jnp.int32),
        pltpu.VMEM((RING, C, DH), jnp.float32),
        pltpu.VMEM((C, DH), jnp.float32),
        pltpu.VMEM((N,), jnp.float32),
        pltpu.VMEM((NS, C), jnp.float32),
        pltpu.VMEM((C,), jnp.float32),
        pltpu.VMEM_SHARED((N, DH), jnp.float32),
        pltpu.VMEM_SHARED((NS, N), jnp.float32),
        [pltpu.SemaphoreType.DMA] * RING,
        [pltpu.SemaphoreType.DMA] * RING,
        pltpu.SemaphoreType.DMA,
    ],
    compiler_params=_sc_params,
)
def _agg(haug_hbm, srcr_hbm, dstr_hbm, out_hbm, src_v, dst_v, gbufs, obuf,
         hist, cntbuf, invbuf, acc, cstage, gsems, ssems, csem):
    c = lax.axis_index("c")
    s = lax.axis_index("s")
    tab = haug_hbm.at[c]

    zero = jnp.zeros((16,), jnp.float32)
    one = jnp.ones((16,), jnp.float32)

    @pl.loop(0, N // 80)
    def _zh(i):
        for j in range(5):
            hist[pl.ds(i * 80 + j * 16, 16)] = zero

    @pl.loop(0, C)
    def _zf(i):
        for j in range(DH // 16):
            gbufs[0, i, pl.ds(j * 16, 16)] = zero

    for k in range(ZPT):
        zc = k * NS + s

        @pl.when(zc < NZC)
        def _zc():
            pltpu.sync_copy(gbufs.at[0], acc.at[pl.ds(zc * C, C)])

    plsc.subcore_barrier()

    def start_gather(j, r):
        pltpu.async_copy(tab.at[src_v.at[j]], gbufs.at[r], gsems[r])

    def wait_gather(j, r):
        pltpu.make_async_copy(tab.at[src_v.at[j]], gbufs.at[r], gsems[r]).wait()

    def start_scatter(j, r):
        pltpu.async_copy(gbufs.at[r], acc.at[dst_v.at[j]], ssems[r], add=True)

    def wait_scatter(j, r):
        pltpu.make_async_copy(gbufs.at[r], acc.at[dst_v.at[j]], ssems[r]).wait()

    for p in range(NPH):
        pltpu.sync_copy(srcr_hbm.at[s, p], src_v)
        pltpu.sync_copy(dstr_hbm.at[s, p], dst_v)

        start_gather(0, 0)
        start_gather(1, 1)

        @pl.loop(0, RITER)
        def _ring(i):
            for r in range(RING):
                j = i * RING + r
                rn = (r + 2) % RING
                if r >= 3:
                    wait_scatter(j - 3, rn)
                else:
                    @pl.when(i > 0)
                    def _ws():
                        wait_scatter(j - 3, rn)
                if r <= 2:
                    start_gather(j + 2, rn)
                else:
                    @pl.when(i < RITER - 1)
                    def _sg():
                        start_gather(j + 2, rn)
                wait_gather(j, r)
                start_scatter(j, r)
                for q in range(C // 16):
                    dv = dst_v[j, pl.ds(q * 16, 16)]
                    plsc.addupdate_scatter(hist, [dv], one)

        for j in (NCH - 3, NCH - 2, NCH - 1):
            wait_scatter(j, j % RING)

    pltpu.sync_copy(hist, cstage.at[s])
    plsc.subcore_barrier()

    for k in range(ZPT):
        zc = k * NS + s

        @pl.when(zc < NZC)
        def _ro():
            for t in range(NS):
                pltpu.async_copy(cstage.at[t, pl.ds(zc * C, C)], cntbuf.at[t], csem)
            pltpu.sync_copy(acc.at[pl.ds(zc * C, C)], gbufs.at[0])
            for t in range(NS):
                pltpu.make_async_copy(
                    cstage.at[t, pl.ds(zc * C, C)], cntbuf.at[t], csem
                ).wait()
            for cb in range(C // 16):
                tot = cntbuf[0, pl.ds(cb * 16, 16)]
                for t in range(1, NS):
                    tot = tot + cntbuf[t, pl.ds(cb * 16, 16)]
                invbuf[pl.ds(cb * 16, 16)] = one / jnp.maximum(tot, one)

            @pl.loop(0, C // 16)
            def _div(g):
                iv16 = invbuf[pl.ds(g * 16, 16)]
                for dr in range(16):
                    inv = iv16[dr]
                    r = g * 16 + dr
                    for cb in range(DH // 16):
                        v = gbufs[0, r, pl.ds(cb * 16, 16)]
                        obuf[r, pl.ds(cb * 16, 16)] = v * inv

            pltpu.sync_copy(obuf, out_hbm.at[pl.ds(zc * C, C), pl.ds(c * DH, DH)])


def kernel(x, W, b, edge_index):
    haug = _mlp(x, W.T, b.reshape(1, D))
    h = jnp.concatenate([haug[0], haug[1]], axis=1)
    srcr = edge_index[0].reshape(NS, NPH, NCH, C)
    dstr = edge_index[1].reshape(NS, NPH, NCH, C)
    mean_h = _agg(haug, srcr, dstr)
    return (h, mean_h)
